# Initial kernel scaffold; baseline (speedup 1.0000x reference)
import jax
import jax.numpy as jnp
from jax.experimental import pallas as pl


def kernel(x, edge_index, params):
    raise NotImplementedError("write your pallas kernel here")


# speedup vs baseline: 1.0596x; 1.0596x over previous
import functools

import jax
import jax.numpy as jnp
from jax.experimental import pallas as pl

N = 10000
E = 320000
D_IN = 128
HID = 256
HEADS = 8
HD = HID // HEADS

NB = 1000
NBLK = N // NB


def _dot(a, b):
    return jnp.dot(a.astype(jnp.bfloat16), b.astype(jnp.bfloat16),
                   preferred_element_type=jnp.float32)


def _stage_a_body(x_ref, w_ref, aw_ref, h_ref, ab_ref):
    h = _dot(x_ref[...], w_ref[...])
    h_ref[...] = h
    ab_ref[...] = _dot(h, aw_ref[...])


def _stage_a(x, w, aw):
    return pl.pallas_call(
        _stage_a_body,
        grid=(NBLK,),
        in_specs=[
            pl.BlockSpec((NB, D_IN), lambda i: (i, 0)),
            pl.BlockSpec((D_IN, HID), lambda i: (0, 0)),
            pl.BlockSpec((HID, 16), lambda i: (0, 0)),
        ],
        out_specs=[
            pl.BlockSpec((NB, HID), lambda i: (i, 0)),
            pl.BlockSpec((NB, 16), lambda i: (i, 0)),
        ],
        out_shape=[
            jax.ShapeDtypeStruct((N, HID), jnp.float32),
            jax.ShapeDtypeStruct((N, 16), jnp.float32),
        ],
    )(x, w, aw)


def _layer_norm(h, g, b):
    mu = jnp.mean(h, axis=-1, keepdims=True)
    var = jnp.mean((h - mu) * (h - mu), axis=-1, keepdims=True)
    return (h - mu) * jax.lax.rsqrt(var + 1e-5) * g + b


def _stage_c_body(o0_ref, o1_ref, s0_ref, s1_ref, r_ref, bias_ref,
                  lng_ref, lnb_ref,
                  w1a_ref, b1a_ref, g1a_ref, bb1a_ref, w2a_ref, b2a_ref,
                  w1r_ref, b1r_ref, g1r_ref, bb1r_ref, w2r_ref, b2r_ref,
                  coords_ref, psum_ref):
    s = s0_ref[...] + s1_ref[...]
    recip = 1.0 / (s + 1e-16)
    rfull = jnp.dot(recip, r_ref[...], preferred_element_type=jnp.float32, precision=jax.lax.Precision.HIGHEST)
    h = jnp.concatenate([o0_ref[...], o1_ref[...]], axis=1) * rfull + bias_ref[...]
    h = _layer_norm(h, lng_ref[...], lnb_ref[...])
    h = jnp.maximum(h, 0.0)
    nrm = jnp.sqrt(jnp.sum(h * h, axis=-1, keepdims=True))
    h = h / jnp.maximum(nrm, 1e-12)

    z = _dot(h, w1a_ref[...]) + b1a_ref[...]
    z = _layer_norm(z, g1a_ref[...], bb1a_ref[...])
    z = jnp.maximum(z, 0.0)
    z2 = _dot(z, w2a_ref[...]) + b2a_ref[...]

    q = _dot(h, w1r_ref[...]) + b1r_ref[...]
    q = _layer_norm(q, g1r_ref[...], bb1r_ref[...])
    q = jnp.maximum(q, 0.0)
    q2 = _dot(q, w2r_ref[...]) + b2r_ref[...]
    rr = jax.nn.sigmoid(q2[:, 0:1])

    r = 0.9 + 0.2 * rr
    theta = z2[:, 0:1]
    phi = z2[:, 1:2]
    st = jnp.sin(theta)
    coords = jnp.concatenate(
        [r * st * jnp.cos(phi), r * st * jnp.sin(phi), r * jnp.cos(theta)], axis=1)
    coords_ref[...] = coords
    psum_ref[...] = jnp.sum(coords, axis=0).reshape(1, 1, 3)


def _stage_c(o0, o1, s0, s1, rmat, bias, lng, lnb, pa, pr):
    full = lambda shape: pl.BlockSpec(shape, lambda i: tuple(0 for _ in shape))
    return pl.pallas_call(
        _stage_c_body,
        grid=(NBLK,),
        in_specs=[
            pl.BlockSpec((NB, 128), lambda i: (i, 0)),
            pl.BlockSpec((NB, 128), lambda i: (i, 0)),
            pl.BlockSpec((NB, HEADS), lambda i: (i, 0)),
            pl.BlockSpec((NB, HEADS), lambda i: (i, 0)),
            full((HEADS, HID)),
            full((1, HID)),
            full((1, HID)),
            full((1, HID)),
            full((HID, HID)), full((1, HID)), full((1, HID)), full((1, HID)),
            full((HID, 2)), full((1, 2)),
            full((HID, 128)), full((1, 128)), full((1, 128)), full((1, 128)),
            full((128, 1)), full((1, 1)),
        ],
        out_specs=[
            pl.BlockSpec((NB, 3), lambda i: (i, 0)),
            pl.BlockSpec((1, 1, 3), lambda i: (i, 0, 0)),
        ],
        out_shape=[
            jax.ShapeDtypeStruct((N, 3), jnp.float32),
            jax.ShapeDtypeStruct((NBLK, 1, 3), jnp.float32),
        ],
    )(o0, o1, s0, s1, rmat, bias, lng, lnb,
      pa['W1'], pa['b1'].reshape(1, -1), pa['g1'].reshape(1, -1), pa['bb1'].reshape(1, -1),
      pa['W2'], pa['b2'].reshape(1, -1),
      pr['W1'], pr['b1'].reshape(1, -1), pr['g1'].reshape(1, -1), pr['bb1'].reshape(1, -1),
      pr['W2'], pr['b2'].reshape(1, -1))


def _stage_d_body(coords_ref, psum_ref, out_ref):
    mean = jnp.sum(psum_ref[...], axis=(0, 1)) * (1.0 / N)
    out_ref[...] = coords_ref[...] - mean.reshape(1, 3)


def _stage_d(coords, psum):
    return pl.pallas_call(
        _stage_d_body,
        out_shape=jax.ShapeDtypeStruct((N, 3), jnp.float32),
    )(coords, psum)


def kernel(x, edge_index, params):
    cp = params['convs'][0]
    src, dst = edge_index[0], edge_index[1]

    eye = jnp.repeat(jnp.eye(HEADS, dtype=jnp.float32), HD, axis=0)
    aw = jnp.concatenate([
        eye * cp['att_src'].reshape(HID, 1),
        eye * cp['att_dst'].reshape(HID, 1),
    ], axis=1)

    h, ab = _stage_a(x, cp['W'], aw)
    a_src, a_dst = ab[:, :HEADS], ab[:, HEADS:]

    e = jax.nn.leaky_relu(a_src[src] + a_dst[dst], 0.2)
    wgt = jnp.exp(e)
    s = jax.ops.segment_sum(wgt, dst, num_segments=N)
    hsh = h.reshape(N, HEADS, HD)
    out = jax.ops.segment_sum(hsh[src] * wgt[:, :, None], dst, num_segments=N)
    o0 = out[:, :4].reshape(N, 128)
    o1 = out[:, 4:].reshape(N, 128)
    s0 = s
    s1 = jnp.zeros_like(s)

    rmat = jnp.repeat(jnp.eye(HEADS, dtype=jnp.float32), HD, axis=1)
    coords, psum = _stage_c(o0, o1, s0, s1, rmat,
                            cp['bias'].reshape(1, HID),
                            cp['ln_g'].reshape(1, HID), cp['ln_b'].reshape(1, HID),
                            params['angle'], params['radius'])
    return _stage_d(coords, psum)


# speedup vs baseline: 36.8266x; 34.7540x over previous
import functools

import jax
import jax.numpy as jnp
from jax import lax
from jax.experimental import pallas as pl
from jax.experimental.pallas import tpu as pltpu
from jax.experimental.pallas import tpu_sc as plsc

N = 10000
E = 320000
D_IN = 128
HID = 256
HEADS = 8
HD = HID // HEADS

NB = 1000
NBLK = N // NB


def _dot(a, b):
    return jnp.dot(a.astype(jnp.bfloat16), b.astype(jnp.bfloat16),
                   preferred_element_type=jnp.float32)


def _stage_a_body(x_ref, w_ref, aws_ref, awd_ref, h2_ref, abs_ref, abd_ref):
    h = _dot(x_ref[...], w_ref[...])
    h2_ref[0] = h[:, :128]
    h2_ref[1] = h[:, 128:]
    abs_ref[...] = _dot(h, aws_ref[...])
    abd_ref[...] = _dot(h, awd_ref[...])


def _stage_a(x, w, aws, awd):
    return pl.pallas_call(
        _stage_a_body,
        grid=(NBLK,),
        in_specs=[
            pl.BlockSpec((NB, D_IN), lambda i: (i, 0)),
            pl.BlockSpec((D_IN, HID), lambda i: (0, 0)),
            pl.BlockSpec((HID, 16), lambda i: (0, 0)),
            pl.BlockSpec((HID, 16), lambda i: (0, 0)),
        ],
        out_specs=[
            pl.BlockSpec((2, NB, 128), lambda i: (0, i, 0)),
            pl.BlockSpec((NB, 16), lambda i: (i, 0)),
            pl.BlockSpec((NB, 16), lambda i: (i, 0)),
        ],
        out_shape=[
            jax.ShapeDtypeStruct((2, N, 128), jnp.float32),
            jax.ShapeDtypeStruct((N, 16), jnp.float32),
            jax.ShapeDtypeStruct((N, 16), jnp.float32),
        ],
    )(x, w, aws, awd)


EB = 128
EPT = 20096
NCHUNK = EPT // EB
NP = N + 8
RPT = 624
TAIL = N - 16 * RPT
ZTAIL = NP - 16 * RPT


def _sc_edge_kernel(h2, absrc, abdst, src, dst):
    mesh = plsc.VectorSubcoreMesh(core_axis_name="c", subcore_axis_name="s")

    @functools.partial(
        pl.kernel,
        mesh=mesh,
        compiler_params=pltpu.CompilerParams(use_tc_tiling_on_sc=False),
        out_type=[
            jax.ShapeDtypeStruct((2 * N, 128), jnp.float32),
            jax.ShapeDtypeStruct((2 * N, 16), jnp.float32),
        ],
        scratch_types=[
            pltpu.VMEM((1, EB), jnp.int32),
            pltpu.VMEM((1, EB), jnp.int32),
            pltpu.VMEM((1, EB), jnp.int32),
            pltpu.VMEM((EB, 16), jnp.float32),
            pltpu.VMEM((EB, 16), jnp.float32),
            pltpu.VMEM((EB, 16), jnp.float32),
            pltpu.VMEM((EB, 128), jnp.float32),
            pltpu.VMEM_SHARED((NP, 128), jnp.float32),
            pltpu.VMEM_SHARED((NP, 16), jnp.float32),
            pltpu.SemaphoreType.DMA,
            pltpu.SemaphoreType.DMA,
        ],
    )
    def body(h2_hbm, absrc_hbm, abdst_hbm, src_hbm, dst_hbm,
             bigout_hbm, sout_hbm,
             src_v, dst_v, srcoff_v, srows_v, drows_v, w_v, hrow_v,
             accum, s_accum, sem_a, sem_h):
        c = lax.axis_index("c")
        t = lax.axis_index("s")
        cN = c * N

        zv = jnp.zeros((16,), jnp.float32)

        def _z128(i, _):
            for d in range(8):
                hrow_v[i, pl.ds(d * 16, 16)] = zv
            return 0
        lax.fori_loop(0, EB, _z128, 0)

        def _z16(i, _):
            w_v[i, :] = zv
            return 0
        lax.fori_loop(0, EB, _z16, 0, unroll=4)

        r0 = t * RPT
        for rr in range(0, RPT, EB):
            sz = min(EB, RPT - rr)
            pltpu.sync_copy(hrow_v.at[pl.ds(0, sz)],
                            accum.at[pl.ds(r0 + rr, sz)])
            pltpu.sync_copy(w_v.at[pl.ds(0, sz)],
                            s_accum.at[pl.ds(r0 + rr, sz)])

        @pl.when(t == 15)
        def _zero_tail():
            pltpu.sync_copy(hrow_v.at[pl.ds(0, ZTAIL)],
                            accum.at[pl.ds(16 * RPT, ZTAIL)])
            pltpu.sync_copy(w_v.at[pl.ds(0, ZTAIL)],
                            s_accum.at[pl.ds(16 * RPT, ZTAIL)])

        plsc.subcore_barrier()

        def chunk(k, _):
            base = t * EPT + k * EB
            pltpu.sync_copy(src_hbm.at[pl.ds(base, EB)], src_v.at[0])
            pltpu.sync_copy(dst_hbm.at[pl.ds(base, EB)], dst_v.at[0])

            def _off(g, _):
                sl = pl.ds(g * 16, 16)
                srcoff_v[0, sl] = src_v[0, sl] + cN
                return 0
            lax.fori_loop(0, EB // 16, _off, 0, unroll=4)

            cp_s = pltpu.async_copy(absrc_hbm.at[src_v.at[0]], srows_v, sem_a)
            cp_d = pltpu.async_copy(abdst_hbm.at[dst_v.at[0]], drows_v, sem_a)
            cp_h = pltpu.async_copy(h2_hbm.at[srcoff_v.at[0]], hrow_v, sem_h)
            cp_s.wait()
            cp_d.wait()

            def _w(g, _):
                v = srows_v[g, :] + drows_v[g, :]
                w_v[g, :] = jnp.exp(jnp.maximum(v, 0.2 * v))
                return 0
            lax.fori_loop(0, EB, _w, 0, unroll=2)

            cp_h.wait()

            col0 = 4 * c
            dnums = lax.GatherDimensionNumbers(
                offset_dims=(), collapsed_slice_dims=(0,), start_index_map=(0,))

            def _scale(i, _):
                wrow = w_v[i, :]
                for j in range(4):
                    idx = jnp.full((16, 1), col0 + j, jnp.int32)
                    wsp = lax.gather(wrow, idx, dnums, (1,),
                                     mode=lax.GatherScatterMode.PROMISE_IN_BOUNDS)
                    for d in range(2):
                        sl = pl.ds(j * 32 + d * 16, 16)
                        hrow_v[i, sl] = hrow_v[i, sl] * wsp
                return 0
            lax.fori_loop(0, EB, _scale, 0)

            pltpu.sync_copy(hrow_v, accum.at[dst_v.at[0]], add=True)

            @pl.when(k % 2 == c)
            def _s_scatter():
                pltpu.sync_copy(w_v, s_accum.at[dst_v.at[0]], add=True)
            return 0

        lax.fori_loop(0, NCHUNK, chunk, 0)
        plsc.subcore_barrier()

        pltpu.sync_copy(accum.at[pl.ds(r0, RPT)],
                        bigout_hbm.at[pl.ds(cN + r0, RPT)])
        pltpu.sync_copy(s_accum.at[pl.ds(r0, RPT)],
                        sout_hbm.at[pl.ds(cN + r0, RPT)])

        @pl.when(t == 15)
        def _read_tail():
            pltpu.sync_copy(accum.at[pl.ds(16 * RPT, TAIL)],
                            bigout_hbm.at[pl.ds(cN + 16 * RPT, TAIL)])
            pltpu.sync_copy(s_accum.at[pl.ds(16 * RPT, TAIL)],
                            sout_hbm.at[pl.ds(cN + 16 * RPT, TAIL)])

    return body(h2, absrc, abdst, src, dst)


def _layer_norm(h, g, b):
    mu = jnp.mean(h, axis=-1, keepdims=True)
    var = jnp.mean((h - mu) * (h - mu), axis=-1, keepdims=True)
    return (h - mu) * jax.lax.rsqrt(var + 1e-5) * g + b


def _stage_c_body(o0_ref, o1_ref, s0_ref, s1_ref, r_ref, bias_ref,
                  lng_ref, lnb_ref,
                  w1a_ref, b1a_ref, g1a_ref, bb1a_ref, w2a_ref, b2a_ref,
                  w1r_ref, b1r_ref, g1r_ref, bb1r_ref, w2r_ref, b2r_ref,
                  coords_ref, psum_ref):
    s = s0_ref[...] + s1_ref[...]
    recip = 1.0 / (s + 1e-16)
    rfull = jnp.dot(recip, r_ref[...], preferred_element_type=jnp.float32, precision=jax.lax.Precision.HIGHEST)
    h = jnp.concatenate([o0_ref[...], o1_ref[...]], axis=1) * rfull + bias_ref[...]
    h = _layer_norm(h, lng_ref[...], lnb_ref[...])
    h = jnp.maximum(h, 0.0)
    nrm = jnp.sqrt(jnp.sum(h * h, axis=-1, keepdims=True))
    h = h / jnp.maximum(nrm, 1e-12)

    z = _dot(h, w1a_ref[...]) + b1a_ref[...]
    z = _layer_norm(z, g1a_ref[...], bb1a_ref[...])
    z = jnp.maximum(z, 0.0)
    z2 = _dot(z, w2a_ref[...]) + b2a_ref[...]

    q = _dot(h, w1r_ref[...]) + b1r_ref[...]
    q = _layer_norm(q, g1r_ref[...], bb1r_ref[...])
    q = jnp.maximum(q, 0.0)
    q2 = _dot(q, w2r_ref[...]) + b2r_ref[...]
    rr = jax.nn.sigmoid(q2[:, 0:1])

    r = 0.9 + 0.2 * rr
    theta = z2[:, 0:1]
    phi = z2[:, 1:2]
    st = jnp.sin(theta)
    coords = jnp.concatenate(
        [r * st * jnp.cos(phi), r * st * jnp.sin(phi), r * jnp.cos(theta)], axis=1)
    coords_ref[...] = coords
    psum_ref[...] = jnp.sum(coords, axis=0).reshape(1, 1, 3)


def _stage_c(o0, o1, s0, s1, rmat, bias, lng, lnb, pa, pr):
    full = lambda shape: pl.BlockSpec(shape, lambda i: tuple(0 for _ in shape))
    return pl.pallas_call(
        _stage_c_body,
        grid=(NBLK,),
        in_specs=[
            pl.BlockSpec((NB, 128), lambda i: (i, 0)),
            pl.BlockSpec((NB, 128), lambda i: (i, 0)),
            pl.BlockSpec((NB, HEADS), lambda i: (i, 0)),
            pl.BlockSpec((NB, HEADS), lambda i: (i, 0)),
            full((HEADS, HID)),
            full((1, HID)),
            full((1, HID)),
            full((1, HID)),
            full((HID, HID)), full((1, HID)), full((1, HID)), full((1, HID)),
            full((HID, 2)), full((1, 2)),
            full((HID, 128)), full((1, 128)), full((1, 128)), full((1, 128)),
            full((128, 1)), full((1, 1)),
        ],
        out_specs=[
            pl.BlockSpec((NB, 3), lambda i: (i, 0)),
            pl.BlockSpec((1, 1, 3), lambda i: (i, 0, 0)),
        ],
        out_shape=[
            jax.ShapeDtypeStruct((N, 3), jnp.float32),
            jax.ShapeDtypeStruct((NBLK, 1, 3), jnp.float32),
        ],
    )(o0, o1, s0, s1, rmat, bias, lng, lnb,
      pa['W1'], pa['b1'].reshape(1, -1), pa['g1'].reshape(1, -1), pa['bb1'].reshape(1, -1),
      pa['W2'], pa['b2'].reshape(1, -1),
      pr['W1'], pr['b1'].reshape(1, -1), pr['g1'].reshape(1, -1), pr['bb1'].reshape(1, -1),
      pr['W2'], pr['b2'].reshape(1, -1))


def _stage_d_body(coords_ref, psum_ref, out_ref):
    mean = jnp.sum(psum_ref[...], axis=(0, 1)) * (1.0 / N)
    out_ref[...] = coords_ref[...] - mean.reshape(1, 3)


def _stage_d(coords, psum):
    return pl.pallas_call(
        _stage_d_body,
        out_shape=jax.ShapeDtypeStruct((N, 3), jnp.float32),
    )(coords, psum)


def kernel(x, edge_index, params):
    cp = params['convs'][0]
    src, dst = edge_index[0], edge_index[1]

    eye = jnp.repeat(jnp.eye(HEADS, dtype=jnp.float32), HD, axis=0)
    es = eye * cp['att_src'].reshape(HID, 1)
    ed = eye * cp['att_dst'].reshape(HID, 1)
    aws = jnp.concatenate([es, es], axis=1)
    awd = jnp.concatenate([ed, ed], axis=1)

    hsplit, absrc, abdst = _stage_a(x, cp['W'], aws, awd)
    h2 = hsplit.reshape(2 * N, 128)

    pad = EPT - E // 16
    src_p = jnp.concatenate(
        [src.reshape(16, E // 16),
         jnp.zeros((16, pad), jnp.int32)], axis=1).reshape(-1)
    dst_p = jnp.concatenate(
        [dst.reshape(16, E // 16),
         jnp.full((16, pad), N, jnp.int32)], axis=1).reshape(-1)

    bigout, sout = _sc_edge_kernel(h2, absrc, abdst, src_p, dst_p)
    o0 = bigout[:N]
    o1 = bigout[N:]
    s0 = sout[:N, :HEADS]
    s1 = sout[N:, :HEADS]

    rmat = jnp.repeat(jnp.eye(HEADS, dtype=jnp.float32), HD, axis=1)
    coords, psum = _stage_c(o0, o1, s0, s1, rmat,
                            cp['bias'].reshape(1, HID),
                            cp['ln_g'].reshape(1, HID), cp['ln_b'].reshape(1, HID),
                            params['angle'], params['radius'])
    return _stage_d(coords, psum)
